# baseline (speedup 1.0000x reference)
import jax
import jax.numpy as jnp
from jax import lax
from jax.experimental import pallas as pl
from jax.experimental.pallas import tpu as pltpu
from jax.experimental.pallas import tpu_sc as plsc

B = 4096
L = 50
PQ_M = 8
VALS_PER_DIM = 256
SUB_DIM = 16

N_TOKENS = B * L
NC = 2
NS = 16
NW = NC * NS
TOK_PER_W = N_TOKENS // NW
CHUNK = 320
ITERS = TOK_PER_W // CHUNK
PAIRS = ITERS // 2
ROWS = CHUNK * PQ_M
NUM_ROWS_HALF = 500001
CSPLIT = 4
CSUB = CHUNK // CSPLIT
GSPLIT = 4
GSUB = ROWS // GSPLIT


def _body(ids_hbm, codes_hbm, ctable_hbm, out_hbm,
          ids_v, codes0, codes1, fidx0, fidx1, out0, out1,
          sc0, sc1, sg0, sg1, ss0, ss1):
    wid = lax.axis_index("s") * NC + lax.axis_index("c")
    tok0 = wid * TOK_PER_W
    pltpu.sync_copy(ids_hbm.at[pl.ds(tok0, TOK_PER_W)], ids_v)
    for i in range(TOK_PER_W // 16):
        ids_v[pl.ds(i * 16, 16)] = ids_v[pl.ds(i * 16, 16)] & 2047

    codes = (codes0, codes1)
    fidx = (fidx0, fidx1)
    out = (out0, out1)
    sc = (sc0, sc1)
    sg = (sg0, sg1)
    ss = (ss0, ss1)

    lanes = lax.iota(jnp.int32, 16)
    row_half = lanes // 8
    col = lanes - row_half * 8
    col_off = col * VALS_PER_DIM

    def codes_copies(g, b):
        for s in range(CSPLIT):
            yield pltpu.make_async_copy(
                codes_hbm.at[ids_v.at[pl.ds(g * CHUNK + s * CSUB, CSUB)]],
                codes[b].at[pl.ds(s * CSUB, CSUB)], sc[b])

    def start_codes(g, b):
        for cp in codes_copies(g, b):
            cp.start()

    def wait_codes(g, b):
        for cp in codes_copies(g, b):
            cp.wait()

    start_codes(0, 0)

    def pair_body(p, carry):
        for b in range(2):
            g = 2 * p + b
            if b == 0:
                start_codes(g + 1, 1)
            else:
                @pl.when(p < PAIRS - 1)
                def _():
                    start_codes(g + 1, 0)
            wait_codes(g, b)
            for i in range(ROWS // 16):
                v = plsc.load_gather(codes[b], [row_half + 2 * i, col])
                fidx[b][pl.ds(i * 16, 16)] = v + col_off
            @pl.when(p >= 1)
            def _():
                pltpu.make_async_copy(
                    out[b],
                    out_hbm.at[pl.ds((tok0 + (g - 2) * CHUNK) * PQ_M, ROWS)],
                    ss[b]).wait()
            gcopies = [
                pltpu.make_async_copy(
                    ctable_hbm.at[fidx[b].at[pl.ds(s * GSUB, GSUB)]],
                    out[b].at[pl.ds(s * GSUB, GSUB)], sg[b])
                for s in range(GSPLIT)
            ]
            for cp in gcopies:
                cp.start()
            for cp in gcopies:
                cp.wait()
            pltpu.async_copy(
                out[b],
                out_hbm.at[pl.ds((tok0 + g * CHUNK) * PQ_M, ROWS)],
                ss[b])
        return carry

    lax.fori_loop(0, PAIRS, pair_body, 0)

    for b in range(2):
        g = ITERS - 2 + b
        pltpu.make_async_copy(
            out[b],
            out_hbm.at[pl.ds((tok0 + g * CHUNK) * PQ_M, ROWS)],
            ss[b]).wait()


@jax.jit
def kernel(input_ids, item_codes, centroids):
    ids_flat = input_ids.reshape(-1)
    ctable = centroids.reshape(PQ_M * VALS_PER_DIM, SUB_DIM)
    item_codes = item_codes.reshape((NUM_ROWS_HALF, 2 * PQ_M))
    mesh = plsc.VectorSubcoreMesh(core_axis_name="c", subcore_axis_name="s")
    out = pl.kernel(
        _body,
        out_type=jax.ShapeDtypeStruct((N_TOKENS * PQ_M, SUB_DIM), jnp.float32),
        mesh=mesh,
        compiler_params=pltpu.CompilerParams(
            use_tc_tiling_on_sc=False, needs_layout_passes=False),
        scratch_types=[
            pltpu.VMEM((TOK_PER_W,), jnp.int32),
            pltpu.VMEM((CHUNK, 2 * PQ_M), jnp.int32),
            pltpu.VMEM((CHUNK, 2 * PQ_M), jnp.int32),
            pltpu.VMEM((ROWS,), jnp.int32),
            pltpu.VMEM((ROWS,), jnp.int32),
            pltpu.VMEM((ROWS, SUB_DIM), jnp.float32),
            pltpu.VMEM((ROWS, SUB_DIM), jnp.float32),
            pltpu.SemaphoreType.DMA,
            pltpu.SemaphoreType.DMA,
            pltpu.SemaphoreType.DMA,
            pltpu.SemaphoreType.DMA,
            pltpu.SemaphoreType.DMA,
            pltpu.SemaphoreType.DMA,
        ],
    )(ids_flat, item_codes, ctable)
    return out.reshape(B, L, PQ_M * SUB_DIM)


# speedup vs baseline: 1.0313x; 1.0313x over previous
import jax
import jax.numpy as jnp
from jax import lax
from jax.experimental import pallas as pl
from jax.experimental.pallas import tpu as pltpu
from jax.experimental.pallas import tpu_sc as plsc

B = 4096
L = 50
PQ_M = 8
VALS_PER_DIM = 256
SUB_DIM = 16

N_TOKENS = B * L
NC = 2
NS = 16
NW = NC * NS
TOK_PER_W = N_TOKENS // NW
CHUNK = 128
ITERS = TOK_PER_W // CHUNK
PAIRS = ITERS // 2
ROWS = CHUNK * PQ_M


def _body(ids_hbm, codes_hbm, ctable_hbm, out_hbm,
          ids_v, codes0, codes1, fidx0, fidx1, out0, out1,
          sc0, sc1, sg0, sg1, ss0, ss1):
    wid = lax.axis_index("s") * NC + lax.axis_index("c")
    tok0 = wid * TOK_PER_W
    pltpu.sync_copy(ids_hbm.at[wid], ids_v)

    codes = (codes0, codes1)
    fidx = (fidx0, fidx1)
    out = (out0, out1)
    sc = (sc0, sc1)
    sg = (sg0, sg1)
    ss = (ss0, ss1)

    lanes = lax.iota(jnp.int32, 16)
    row_half = lanes // 8
    col = lanes - row_half * 8
    col_off = col * VALS_PER_DIM

    def start_codes(g, b):
        pltpu.async_copy(codes_hbm.at[ids_v.at[g]], codes[b], sc[b])

    def wait_codes(g, b):
        pltpu.make_async_copy(
            codes_hbm.at[ids_v.at[g]], codes[b], sc[b]).wait()

    start_codes(0, 0)

    def pair_body(p, carry):
        for b in range(2):
            g = 2 * p + b
            if b == 0:
                start_codes(g + 1, 1)
            else:
                @pl.when(p < PAIRS - 1)
                def _():
                    start_codes(g + 1, 0)
            wait_codes(g, b)
            for i in range(ROWS // 16):
                v = plsc.load_gather(codes[b], [row_half + 2 * i, col])
                fidx[b][pl.ds(i * 16, 16)] = v + col_off
            @pl.when(p >= 1)
            def _():
                pltpu.make_async_copy(
                    out[b],
                    out_hbm.at[pl.ds((tok0 + (g - 2) * CHUNK) * PQ_M, ROWS)],
                    ss[b]).wait()
            pltpu.async_copy(ctable_hbm.at[fidx[b]], out[b], sg[b]).wait()
            pltpu.async_copy(
                out[b],
                out_hbm.at[pl.ds((tok0 + g * CHUNK) * PQ_M, ROWS)],
                ss[b])
        return carry

    lax.fori_loop(0, PAIRS, pair_body, 0)

    for b in range(2):
        g = ITERS - 2 + b
        pltpu.make_async_copy(
            out[b],
            out_hbm.at[pl.ds((tok0 + g * CHUNK) * PQ_M, ROWS)],
            ss[b]).wait()


@jax.jit
def kernel(input_ids, item_codes, centroids):
    ids_blocked = input_ids.reshape(NW, ITERS, CHUNK)
    ctable = centroids.reshape(PQ_M * VALS_PER_DIM, SUB_DIM)
    mesh = plsc.VectorSubcoreMesh(core_axis_name="c", subcore_axis_name="s")
    out = pl.kernel(
        _body,
        out_type=jax.ShapeDtypeStruct((N_TOKENS * PQ_M, SUB_DIM), jnp.float32),
        mesh=mesh,
        compiler_params=pltpu.CompilerParams(
            use_tc_tiling_on_sc=False, needs_layout_passes=False),
        scratch_types=[
            pltpu.VMEM((ITERS, CHUNK), jnp.int32),
            pltpu.VMEM((CHUNK, PQ_M), jnp.int32),
            pltpu.VMEM((CHUNK, PQ_M), jnp.int32),
            pltpu.VMEM((ROWS,), jnp.int32),
            pltpu.VMEM((ROWS,), jnp.int32),
            pltpu.VMEM((ROWS, SUB_DIM), jnp.float32),
            pltpu.VMEM((ROWS, SUB_DIM), jnp.float32),
            pltpu.SemaphoreType.DMA,
            pltpu.SemaphoreType.DMA,
            pltpu.SemaphoreType.DMA,
            pltpu.SemaphoreType.DMA,
            pltpu.SemaphoreType.DMA,
            pltpu.SemaphoreType.DMA,
        ],
    )(ids_blocked, item_codes, ctable)
    return out.reshape(B, L, PQ_M * SUB_DIM)
